# Initial kernel scaffold; baseline (speedup 1.0000x reference)
import jax
import jax.numpy as jnp
from jax.experimental import pallas as pl


def kernel(xyz):
    raise NotImplementedError("write your pallas kernel here")


# speedup vs baseline: 28.4004x; 28.4004x over previous
import jax
import jax.numpy as jnp
from jax.experimental import pallas as pl
from jax.experimental.pallas import tpu as pltpu

KNN = 16
NFPS = 256
B = 8
N = 2048
ROW_TILE = 256
N_TILES = N // ROW_TILE


def _knn_moments_kernel(xt_ref, xT_ref, mask_ref, xmean_ref, c6_ref):
    xt = xt_ref[0]
    xT = xT_ref[0]
    s1 = jnp.sum(xt * xt, axis=1, keepdims=True)
    s2 = jnp.sum(xT * xT, axis=0, keepdims=True)
    dot = jnp.dot(xt, xT, preferred_element_type=jnp.float32)
    D = (s1 - 2.0 * dot) + s2

    iota = jax.lax.broadcasted_iota(jnp.int32, D.shape, 1)
    zero = jnp.float32(0.0)
    x0b = jnp.broadcast_to(xT[0:1, :], D.shape)
    x1b = jnp.broadcast_to(xT[1:2, :], D.shape)
    x2b = jnp.broadcast_to(xT[2:3, :], D.shape)
    mask_acc = jnp.zeros_like(D)
    cols = []
    for _ in range(KNN):
        am = jnp.argmin(D, axis=1).astype(jnp.int32)[:, None]
        hot = iota == am
        n0 = jnp.sum(jnp.where(hot, x0b, zero), axis=1, keepdims=True)
        n1 = jnp.sum(jnp.where(hot, x1b, zero), axis=1, keepdims=True)
        n2 = jnp.sum(jnp.where(hot, x2b, zero), axis=1, keepdims=True)
        cols.append((n0, n1, n2))
        mask_acc = mask_acc + hot.astype(jnp.float32)
        D = jnp.where(hot, jnp.float32(float('inf')), D)
    mask_ref[0] = mask_acc

    inv_k = jnp.float32(1.0 / KNN)
    s0 = cols[0][0]
    s1c = cols[0][1]
    s2c = cols[0][2]
    for r in range(1, KNN):
        s0 = s0 + cols[r][0]
        s1c = s1c + cols[r][1]
        s2c = s2c + cols[r][2]
    m0 = s0 * inv_k
    m1 = s1c * inv_k
    m2 = s2c * inv_k
    xmean_ref[0, :, 0:1] = m0
    xmean_ref[0, :, 1:2] = m1
    xmean_ref[0, :, 2:3] = m2

    c00 = jnp.zeros_like(m0)
    c01 = jnp.zeros_like(m0)
    c02 = jnp.zeros_like(m0)
    c11 = jnp.zeros_like(m0)
    c12 = jnp.zeros_like(m0)
    c22 = jnp.zeros_like(m0)
    for r in range(KNN):
        d0 = (cols[r][0] - m0).astype(jnp.bfloat16).astype(jnp.float32)
        d1 = (cols[r][1] - m1).astype(jnp.bfloat16).astype(jnp.float32)
        d2 = (cols[r][2] - m2).astype(jnp.bfloat16).astype(jnp.float32)
        c00 = c00 + d0 * d0
        c01 = c01 + d0 * d1
        c02 = c02 + d0 * d2
        c11 = c11 + d1 * d1
        c12 = c12 + d1 * d2
        c22 = c22 + d2 * d2
    inv_km1 = jnp.float32(1.0 / (KNN - 1))
    c6_ref[0, :, 0:1] = c00 * inv_km1
    c6_ref[0, :, 1:2] = c01 * inv_km1
    c6_ref[0, :, 2:3] = c02 * inv_km1
    c6_ref[0, :, 3:4] = c11 * inv_km1
    c6_ref[0, :, 4:5] = c12 * inv_km1
    c6_ref[0, :, 5:6] = c22 * inv_km1


_JACOBI_ORDER = ((0, 2), (1, 2), (0, 1))
_SWEEPS = 4


def _eigh_kernel(c6_ref, nrm_ref):
    a = {
        (0, 0): c6_ref[0], (0, 1): c6_ref[1], (0, 2): c6_ref[2],
        (1, 1): c6_ref[3], (1, 2): c6_ref[4], (2, 2): c6_ref[5],
    }
    one = jnp.float32(1.0)
    two = jnp.float32(2.0)
    eye = jnp.ones_like(a[(0, 0)])
    zero = jnp.zeros_like(a[(0, 0)])
    v = {}
    for i in range(3):
        for j in range(3):
            v[(i, j)] = eye if i == j else zero

    def at(d, i, j):
        return d[(i, j)] if i <= j else d[(j, i)]

    def put(d, i, j, val):
        if i <= j:
            d[(i, j)] = val
        else:
            d[(j, i)] = val

    for _ in range(_SWEEPS):
        for (p, q) in _JACOBI_ORDER:
            r = 3 - p - q
            apq = at(a, p, q)
            app = at(a, p, p)
            aqq = at(a, q, q)
            apr = at(a, p, r)
            aqr = at(a, q, r)
            tau = (aqq - app) / (two * apq)
            sg = jnp.where(tau >= 0.0, one, -one)
            t = sg / (jnp.abs(tau) + jnp.sqrt(one + tau * tau))
            t = jnp.where(apq == 0.0, jnp.float32(0.0), t)
            c = one / jnp.sqrt(one + t * t)
            s = t * c
            put(a, p, p, c * c * app - two * s * c * apq + s * s * aqq)
            put(a, q, q, s * s * app + two * s * c * apq + c * c * aqq)
            put(a, p, q, zero)
            put(a, p, r, c * apr - s * aqr)
            put(a, q, r, s * apr + c * aqr)
            for i in range(3):
                vp = v[(i, p)]
                vq = v[(i, q)]
                v[(i, p)] = c * vp - s * vq
                v[(i, q)] = s * vp + c * vq

    l0, l1, l2 = a[(0, 0)], a[(1, 1)], a[(2, 2)]
    pick0 = (l0 <= l1) & (l0 <= l2)
    pick1 = l1 <= l2
    n0 = jnp.where(pick0, v[(0, 0)], jnp.where(pick1, v[(0, 1)], v[(0, 2)]))
    n1 = jnp.where(pick0, v[(1, 0)], jnp.where(pick1, v[(1, 1)], v[(1, 2)]))
    n2 = jnp.where(pick0, v[(2, 0)], jnp.where(pick1, v[(2, 1)], v[(2, 2)]))
    nrm = jnp.sqrt(n0 * n0 + n1 * n1 + n2 * n2)
    nrm = jnp.maximum(nrm, jnp.float32(1e-12))
    nrm_ref[0] = n0 / nrm
    nrm_ref[1] = n1 / nrm
    nrm_ref[2] = n2 / nrm


def _normal_update_kernel(mask_ref, nall_ref, xt_ref, xmean_ref,
                          xnu_ref, nmean_ref):
    mask = mask_ref[0]
    nall = nall_ref[0]
    x = xt_ref[0]
    xm = xmean_ref[0]
    nsum = jax.lax.dot_general(
        mask, nall, (((1,), (0,)), ((), ())),
        precision=jax.lax.Precision.HIGHEST,
        preferred_element_type=jnp.float32)
    nm = nsum * jnp.float32(1.0 / KNN)
    nn = jnp.sqrt(jnp.sum(nm * nm, axis=1, keepdims=True))
    n = nm / jnp.maximum(nn, jnp.float32(1e-12))
    delta = x - xm

    def bf(v):
        return v.astype(jnp.bfloat16).astype(jnp.float32)

    nc = [n[:, i:i + 1] for i in range(3)]
    db = [bf(delta[:, i:i + 1]) for i in range(3)]
    corr = []
    for i in range(3):
        pn = [bf(nc[i] * nc[j]) for j in range(3)]
        corr.append((pn[0] * db[0] + pn[1] * db[1]) + pn[2] * db[2])
    for i in range(3):
        xnu_ref[0, :, i:i + 1] = x[:, i:i + 1] - corr[i]
    nmean_ref[0] = n


def _fps_kernel(x0_ref, x1_ref, x2_ref, q0_ref, q1_ref, q2_ref):
    x0 = x0_ref[...]
    x1 = x1_ref[...]
    x2 = x2_ref[...]
    iota = jax.lax.broadcasted_iota(jnp.int32, (B, N), 1)
    qiota = jax.lax.broadcasted_iota(jnp.int32, (B, NFPS), 1)
    zero = jnp.float32(0.0)

    def body(t, carry):
        cur, min_d, a0, a1, a2 = carry
        onehot = iota == cur
        p0 = jnp.sum(jnp.where(onehot, x0, zero), axis=1, keepdims=True)
        p1 = jnp.sum(jnp.where(onehot, x1, zero), axis=1, keepdims=True)
        p2 = jnp.sum(jnp.where(onehot, x2, zero), axis=1, keepdims=True)
        slot = qiota == t
        a0 = jnp.where(slot, p0, a0)
        a1 = jnp.where(slot, p1, a1)
        a2 = jnp.where(slot, p2, a2)
        d0 = x0 - p0
        d1 = x1 - p1
        d2 = x2 - p2
        d = (d0 * d0 + d1 * d1) + d2 * d2
        min_d = jnp.minimum(min_d, d)
        cur = jnp.argmax(min_d, axis=1).astype(jnp.int32)[:, None]
        return cur, min_d, a0, a1, a2

    cur0 = jnp.zeros((B, 1), jnp.int32)
    mind0 = jnp.full((B, N), jnp.float32(float('inf')), jnp.float32)
    acc0 = jnp.zeros((B, NFPS), jnp.float32)
    _, _, a0, a1, a2 = jax.lax.fori_loop(
        0, NFPS, body, (cur0, mind0, acc0, acc0, acc0))
    q0_ref[...] = a0
    q1_ref[...] = a1
    q2_ref[...] = a2


def _knn2_kernel(q_ref, cat6_ref, xnuT_ref, out_ref):
    q = q_ref[0]
    cat6 = cat6_ref[0]
    xnuT = xnuT_ref[0]
    s1 = jnp.sum(q * q, axis=1, keepdims=True)
    s2 = jnp.sum(xnuT * xnuT, axis=0, keepdims=True)
    dot = jnp.dot(q, xnuT, preferred_element_type=jnp.float32)
    D = (s1 - 2.0 * dot) + s2

    iota = jax.lax.broadcasted_iota(jnp.int32, D.shape, 1)
    sums = None
    for _ in range(KNN):
        am = jnp.argmin(D, axis=1).astype(jnp.int32)[:, None]
        sel = (iota == am).astype(jnp.float32)
        nbr = jax.lax.dot_general(
            sel, cat6, (((1,), (0,)), ((), ())),
            precision=jax.lax.Precision.HIGHEST,
            preferred_element_type=jnp.float32)
        sums = nbr if sums is None else sums + nbr
        D = jnp.where(sel > 0.0, jnp.float32(float('inf')), D)

    inv_k = jnp.float32(1.0 / KNN)
    xm = sums[:, 0:3] * inv_k
    nm = sums[:, 3:6] * inv_k
    nn = jnp.sqrt(jnp.sum(nm * nm, axis=1, keepdims=True))
    n2 = nm / jnp.maximum(nn, jnp.float32(1e-12))
    delta2 = q - xm

    def bf(v):
        return v.astype(jnp.bfloat16).astype(jnp.float32)

    nc = [n2[:, i:i + 1] for i in range(3)]
    db = [bf(delta2[:, i:i + 1]) for i in range(3)]
    one = jnp.float32(1.0)
    for i in range(3):
        pt = []
        for j in range(3):
            eye = one if i == j else jnp.float32(0.0)
            pt.append(bf(eye - nc[i] * nc[j]))
        corr = (pt[0] * db[0] + pt[1] * db[1]) + pt[2] * db[2]
        out_ref[0, :, i:i + 1] = q[:, i:i + 1] - corr


def kernel(xyz):
    xyz = jnp.asarray(xyz, jnp.float32)
    xT = jnp.transpose(xyz, (0, 2, 1))

    mask, xmean, c6 = pl.pallas_call(
        _knn_moments_kernel,
        grid=(B, N_TILES),
        in_specs=[
            pl.BlockSpec((1, ROW_TILE, 3), lambda b, t: (b, t, 0)),
            pl.BlockSpec((1, 3, N), lambda b, t: (b, 0, 0)),
        ],
        out_specs=[
            pl.BlockSpec((1, ROW_TILE, N), lambda b, t: (b, t, 0)),
            pl.BlockSpec((1, ROW_TILE, 3), lambda b, t: (b, t, 0)),
            pl.BlockSpec((1, ROW_TILE, 6), lambda b, t: (b, t, 0)),
        ],
        out_shape=[
            jax.ShapeDtypeStruct((B, N, N), jnp.float32),
            jax.ShapeDtypeStruct((B, N, 3), jnp.float32),
            jax.ShapeDtypeStruct((B, N, 6), jnp.float32),
        ],
        compiler_params=pltpu.CompilerParams(
            dimension_semantics=("parallel", "parallel")),
    )(xyz, xT)

    c6T = jnp.transpose(c6, (2, 0, 1))
    nrmT = pl.pallas_call(
        _eigh_kernel,
        out_shape=jax.ShapeDtypeStruct((3, B, N), jnp.float32),
    )(c6T)
    normal = jnp.transpose(nrmT, (1, 2, 0))

    xnu, nmean = pl.pallas_call(
        _normal_update_kernel,
        grid=(B, N_TILES),
        in_specs=[
            pl.BlockSpec((1, ROW_TILE, N), lambda b, t: (b, t, 0)),
            pl.BlockSpec((1, N, 3), lambda b, t: (b, 0, 0)),
            pl.BlockSpec((1, ROW_TILE, 3), lambda b, t: (b, t, 0)),
            pl.BlockSpec((1, ROW_TILE, 3), lambda b, t: (b, t, 0)),
        ],
        out_specs=[
            pl.BlockSpec((1, ROW_TILE, 3), lambda b, t: (b, t, 0)),
            pl.BlockSpec((1, ROW_TILE, 3), lambda b, t: (b, t, 0)),
        ],
        out_shape=[
            jax.ShapeDtypeStruct((B, N, 3), jnp.float32),
            jax.ShapeDtypeStruct((B, N, 3), jnp.float32),
        ],
        compiler_params=pltpu.CompilerParams(
            dimension_semantics=("parallel", "parallel")),
    )(mask, normal, xyz, xmean)

    q0, q1, q2 = pl.pallas_call(
        _fps_kernel,
        out_shape=[
            jax.ShapeDtypeStruct((B, NFPS), jnp.float32),
            jax.ShapeDtypeStruct((B, NFPS), jnp.float32),
            jax.ShapeDtypeStruct((B, NFPS), jnp.float32),
        ],
    )(xnu[:, :, 0], xnu[:, :, 1], xnu[:, :, 2])
    q = jnp.stack([q0, q1, q2], axis=-1)

    cat6 = jnp.concatenate([xnu, nmean], axis=-1)
    xnuT = jnp.transpose(xnu, (0, 2, 1))
    out = pl.pallas_call(
        _knn2_kernel,
        grid=(B,),
        in_specs=[
            pl.BlockSpec((1, NFPS, 3), lambda b: (b, 0, 0)),
            pl.BlockSpec((1, N, 6), lambda b: (b, 0, 0)),
            pl.BlockSpec((1, 3, N), lambda b: (b, 0, 0)),
        ],
        out_specs=pl.BlockSpec((1, NFPS, 3), lambda b: (b, 0, 0)),
        out_shape=jax.ShapeDtypeStruct((B, NFPS, 3), jnp.float32),
        compiler_params=pltpu.CompilerParams(
            dimension_semantics=("parallel",)),
    )(q, cat6, xnuT)
    return out
